# baseline (speedup 1.0000x reference)
import functools

import jax
import jax.numpy as jnp
from jax import lax
from jax.experimental import pallas as pl
from jax.experimental.pallas import tpu as pltpu
from jax.experimental.pallas import tpu_sc as plsc

NC = 2
NS = 16
NW = NC * NS
L = 16
CHUNK = 80
BR = 25


def _make_sc_segment_sum(n, npad, d, nb, with_deg):
  rpt = npad // NS

  mesh = plsc.VectorSubcoreMesh(core_axis_name="c", subcore_axis_name="s")

  out_type = [jax.ShapeDtypeStruct((NC, npad, d), jnp.float32)]
  if with_deg:
    out_type.append(jax.ShapeDtypeStruct((NW, npad), jnp.float32))

  assert rpt % CHUNK == 0

  scratch = [
      pltpu.VMEM((BR, CHUNK), jnp.int32),
      pltpu.VMEM((BR, CHUNK), jnp.int32),
      pltpu.VMEM((CHUNK, d), jnp.float32),
      pltpu.VMEM((CHUNK, d), jnp.float32),
      pltpu.VMEM_SHARED((npad, d), jnp.float32),
      pltpu.SemaphoreType.DMA,
      pltpu.SemaphoreType.DMA,
      pltpu.SemaphoreType.DMA,
      pltpu.SemaphoreType.DMA,
  ]
  if with_deg:
    scratch.append(pltpu.VMEM((npad,), jnp.float32))

  def body(x_hbm, src_hbm, dst_hbm, zrows_hbm, zdeg_hbm,
           s_out, deg_out, src_v, dst_v, rows_v0, rows_v1, acc_sh,
           gsem0, gsem1, tsem0, tsem1, degp):
    c = lax.axis_index("c")
    s = lax.axis_index("s")
    wid = c * NS + s

    pltpu.sync_copy(zrows_hbm, rows_v0)

    def zstep(r, carry):
      pltpu.sync_copy(rows_v0, acc_sh.at[pl.ds(s * rpt + r * CHUNK, CHUNK)])
      return carry

    lax.fori_loop(0, rpt // CHUNK, zstep, 0)
    if with_deg:
      pltpu.sync_copy(zdeg_hbm, degp)

    plsc.subcore_barrier()

    ones16 = jnp.full((L,), 1.0, jnp.float32)

    def deg_add(j):
      if with_deg:
        for k in range(CHUNK // L):
          idx = dst_v[j, pl.ds(k * L, L)]
          plsc.addupdate_scatter(degp, [idx], ones16)

    def block(b, carry):
      pltpu.sync_copy(src_hbm.at[wid, b], src_v)
      pltpu.sync_copy(dst_hbm.at[wid, b], dst_v)

      pltpu.async_copy(x_hbm.at[src_v.at[0]], rows_v0, gsem0)

      def pair(j2, carry2):
        j = 2 * j2

        @pl.when(j + 1 < BR)
        def _():
          pltpu.async_copy(x_hbm.at[src_v.at[j + 1]], rows_v1, gsem1)

        pltpu.make_async_copy(x_hbm.at[src_v.at[j]], rows_v0, gsem0).wait()
        pltpu.async_copy(rows_v0, acc_sh.at[dst_v.at[j]], tsem0, add=True)
        deg_add(j)

        @pl.when(j + 1 < BR)
        def _():
          pltpu.make_async_copy(
              x_hbm.at[src_v.at[j + 1]], rows_v1, gsem1).wait()
          pltpu.async_copy(rows_v1, acc_sh.at[dst_v.at[j + 1]], tsem1,
                           add=True)
          deg_add(j + 1)

        pltpu.make_async_copy(
            rows_v0, acc_sh.at[dst_v.at[j]], tsem0).wait()

        @pl.when(j + 2 < BR)
        def _():
          pltpu.async_copy(x_hbm.at[src_v.at[j + 2]], rows_v0, gsem0)

        @pl.when(j + 1 < BR)
        def _():
          pltpu.make_async_copy(
              rows_v1, acc_sh.at[dst_v.at[j + 1]], tsem1).wait()

        return carry2

      lax.fori_loop(0, (BR + 1) // 2, pair, 0)
      return carry

    lax.fori_loop(0, nb, block, 0)

    if with_deg:
      pltpu.sync_copy(degp, deg_out.at[wid])

    plsc.subcore_barrier()

    pltpu.sync_copy(acc_sh.at[pl.ds(s * rpt, rpt)],
                    s_out.at[c, pl.ds(s * rpt, rpt)])

  cp = pltpu.CompilerParams(needs_layout_passes=False)
  if with_deg:
    def body_wd(x_hbm, src_hbm, dst_hbm, zr, zd, s_out, deg_out,
                src_v, dst_v, rows_v0, rows_v1, acc_sh,
                gsem0, gsem1, tsem0, tsem1, degp):
      body(x_hbm, src_hbm, dst_hbm, zr, zd, s_out, deg_out,
           src_v, dst_v, rows_v0, rows_v1, acc_sh,
           gsem0, gsem1, tsem0, tsem1, degp)
    fn = pl.kernel(body_wd, out_type=tuple(out_type), mesh=mesh,
                   compiler_params=cp, scratch_types=scratch)
  else:
    def body_nd(x_hbm, src_hbm, dst_hbm, zr, zd, s_out,
                src_v, dst_v, rows_v0, rows_v1, acc_sh,
                gsem0, gsem1, tsem0, tsem1):
      body(x_hbm, src_hbm, dst_hbm, zr, zd, s_out, None,
           src_v, dst_v, rows_v0, rows_v1, acc_sh,
           gsem0, gsem1, tsem0, tsem1, None)
    fn = pl.kernel(body_nd, out_type=tuple(out_type), mesh=mesh,
                   compiler_params=cp, scratch_types=scratch)
  return fn


def _tc_layer(s0_ref, s1_ref, d_ref, x_ref, wl_ref, bl_ref, wr_ref, o_ref,
              *, bn, last):
  pid = pl.program_id(0)
  deg = jnp.sum(d_ref[:, pl.ds(pid * bn, bn)], axis=0)[:, None]
  agg = (s0_ref[...] + s1_ref[...]) / jnp.maximum(deg, 1.0)
  z = lax.dot_general(agg, wl_ref[...], (((1,), (1,)), ((), ())),
                      preferred_element_type=jnp.float32)
  z = z + bl_ref[...]
  z = z + lax.dot_general(x_ref[...], wr_ref[...], (((1,), (1,)), ((), ())),
                          preferred_element_type=jnp.float32)
  if not last:
    o_ref[...] = jnp.maximum(z, 0.0)
  else:
    m = jnp.max(z, axis=1, keepdims=True)
    lse = jnp.log(jnp.sum(jnp.exp(z - m), axis=1, keepdims=True)) + m
    o_ref[...] = z - lse


def _tc_call(s_parts, deg_parts, x, wl, bl, wr, bn, last):
  n, d = x.shape
  npad = deg_parts.shape[1]
  grid = ((n + bn - 1) // bn,)
  row_spec = pl.BlockSpec((bn, d), lambda i: (i, 0))
  deg_spec = pl.BlockSpec((NW, npad), lambda i: (0, 0))
  w_spec = pl.BlockSpec((d, d), lambda i: (0, 0))
  b_spec = pl.BlockSpec((1, d), lambda i: (0, 0))
  return pl.pallas_call(
      functools.partial(_tc_layer, bn=bn, last=last),
      grid=grid,
      in_specs=[row_spec, row_spec, deg_spec, row_spec,
                w_spec, b_spec, w_spec],
      out_specs=row_spec,
      out_shape=jax.ShapeDtypeStruct((n, d), jnp.float32),
  )(s_parts[0], s_parts[1], deg_parts, x, wl, bl.reshape(1, d), wr)


@jax.jit
def kernel(x, edge_index, Wl1, bl1, Wr1, Wl2, bl2, Wr2):
  n, d = x.shape
  e = edge_index.shape[1]
  epb = BR * CHUNK
  assert e % (NW * epb) == 0
  nb = e // (NW * epb)
  bn = 1024
  npad = ((n + bn - 1) // bn) * bn
  assert npad % (8 * NS) == 0

  ei = edge_index.astype(jnp.int32)
  src4 = ei[0].reshape(NW, nb, BR, CHUNK)
  dst4 = ei[1].reshape(NW, nb, BR, CHUNK)

  zrows = jnp.zeros((CHUNK, d), jnp.float32)
  zdeg = jnp.zeros((npad,), jnp.float32)

  sc_l1 = _make_sc_segment_sum(n, npad, d, nb, with_deg=True)
  sc_l2 = _make_sc_segment_sum(n, npad, d, nb, with_deg=False)

  s1_parts, deg_parts = sc_l1(x, src4, dst4, zrows, zdeg)
  h = _tc_call(s1_parts, deg_parts, x, Wl1, bl1, Wr1, bn, last=False)

  (s2_parts,) = sc_l2(h, src4, dst4, zrows, zdeg)
  out = _tc_call(s2_parts, deg_parts, h, Wl2, bl2, Wr2, bn, last=True)
  return out


# speedup vs baseline: 1.1896x; 1.1896x over previous
import functools

import jax
import jax.numpy as jnp
from jax import lax
from jax.experimental import pallas as pl
from jax.experimental.pallas import tpu as pltpu
from jax.experimental.pallas import tpu_sc as plsc

NC = 2
NS = 16
NW = NC * NS
L = 16
CHUNK = 80
BR = 25


def _make_sc_segment_sum(n, npad, d, nb, with_deg):
  rpt = npad // NS

  mesh = plsc.VectorSubcoreMesh(core_axis_name="c", subcore_axis_name="s")

  out_type = [jax.ShapeDtypeStruct((NC, npad, d), jnp.float32)]
  if with_deg:
    out_type.append(jax.ShapeDtypeStruct((NW, npad), jnp.float32))

  assert rpt % CHUNK == 0

  scratch = [
      pltpu.VMEM((BR, CHUNK), jnp.int32),
      pltpu.VMEM((BR, CHUNK), jnp.int32),
      pltpu.VMEM((CHUNK, d), jnp.float32),
      pltpu.VMEM((CHUNK, d), jnp.float32),
      pltpu.VMEM_SHARED((npad, d), jnp.float32),
      pltpu.SemaphoreType.DMA,
      pltpu.SemaphoreType.DMA,
      pltpu.SemaphoreType.DMA,
      pltpu.SemaphoreType.DMA,
  ]
  if with_deg:
    scratch.append(pltpu.VMEM((npad,), jnp.float32))

  def body(x_hbm, src_hbm, dst_hbm, zrows_hbm, zdeg_hbm,
           s_out, deg_out, src_v, dst_v, rows_v0, rows_v1, acc_sh,
           gsem0, gsem1, tsem0, tsem1, degp):
    c = lax.axis_index("c")
    s = lax.axis_index("s")
    wid = c * NS + s

    pltpu.sync_copy(zrows_hbm, rows_v0)

    def zstep(r, carry):
      pltpu.sync_copy(rows_v0, acc_sh.at[pl.ds(s * rpt + r * CHUNK, CHUNK)])
      return carry

    lax.fori_loop(0, rpt // CHUNK, zstep, 0)
    if with_deg:
      pltpu.sync_copy(zdeg_hbm, degp)

    plsc.subcore_barrier()

    ones16 = jnp.full((L,), 1.0, jnp.float32)

    def deg_add(j):
      if with_deg:
        for k in range(CHUNK // L):
          idx = dst_v[j, pl.ds(k * L, L)]
          plsc.addupdate_scatter(degp, [idx], ones16)

    def block(b, carry):
      pltpu.sync_copy(src_hbm.at[wid, b], src_v)
      pltpu.sync_copy(dst_hbm.at[wid, b], dst_v)

      pltpu.async_copy(x_hbm.at[src_v.at[0]], rows_v0, gsem0)

      def pair(j2, carry2):
        j = 2 * j2

        @pl.when(j + 1 < BR)
        def _():
          pltpu.async_copy(x_hbm.at[src_v.at[j + 1]], rows_v1, gsem1)

        pltpu.make_async_copy(x_hbm.at[src_v.at[j]], rows_v0, gsem0).wait()
        pltpu.sync_copy(rows_v0, acc_sh.at[dst_v.at[j]], add=True)
        deg_add(j)

        @pl.when(j + 2 < BR)
        def _():
          pltpu.async_copy(x_hbm.at[src_v.at[j + 2]], rows_v0, gsem0)

        @pl.when(j + 1 < BR)
        def _():
          pltpu.make_async_copy(
              x_hbm.at[src_v.at[j + 1]], rows_v1, gsem1).wait()
          pltpu.sync_copy(rows_v1, acc_sh.at[dst_v.at[j + 1]], add=True)
          deg_add(j + 1)

        return carry2

      lax.fori_loop(0, (BR + 1) // 2, pair, 0)
      return carry

    lax.fori_loop(0, nb, block, 0)

    if with_deg:
      pltpu.sync_copy(degp, deg_out.at[wid])

    plsc.subcore_barrier()

    pltpu.sync_copy(acc_sh.at[pl.ds(s * rpt, rpt)],
                    s_out.at[c, pl.ds(s * rpt, rpt)])

  cp = pltpu.CompilerParams(needs_layout_passes=False)
  if with_deg:
    def body_wd(x_hbm, src_hbm, dst_hbm, zr, zd, s_out, deg_out,
                src_v, dst_v, rows_v0, rows_v1, acc_sh,
                gsem0, gsem1, tsem0, tsem1, degp):
      body(x_hbm, src_hbm, dst_hbm, zr, zd, s_out, deg_out,
           src_v, dst_v, rows_v0, rows_v1, acc_sh,
           gsem0, gsem1, tsem0, tsem1, degp)
    fn = pl.kernel(body_wd, out_type=tuple(out_type), mesh=mesh,
                   compiler_params=cp, scratch_types=scratch)
  else:
    def body_nd(x_hbm, src_hbm, dst_hbm, zr, zd, s_out,
                src_v, dst_v, rows_v0, rows_v1, acc_sh,
                gsem0, gsem1, tsem0, tsem1):
      body(x_hbm, src_hbm, dst_hbm, zr, zd, s_out, None,
           src_v, dst_v, rows_v0, rows_v1, acc_sh,
           gsem0, gsem1, tsem0, tsem1, None)
    fn = pl.kernel(body_nd, out_type=tuple(out_type), mesh=mesh,
                   compiler_params=cp, scratch_types=scratch)
  return fn


def _tc_layer(s0_ref, s1_ref, d_ref, x_ref, wl_ref, bl_ref, wr_ref, o_ref,
              *, bn, last):
  pid = pl.program_id(0)
  deg = jnp.sum(d_ref[:, pl.ds(pid * bn, bn)], axis=0)[:, None]
  agg = (s0_ref[...] + s1_ref[...]) / jnp.maximum(deg, 1.0)
  z = lax.dot_general(agg, wl_ref[...], (((1,), (1,)), ((), ())),
                      preferred_element_type=jnp.float32)
  z = z + bl_ref[...]
  z = z + lax.dot_general(x_ref[...], wr_ref[...], (((1,), (1,)), ((), ())),
                          preferred_element_type=jnp.float32)
  if not last:
    o_ref[...] = jnp.maximum(z, 0.0)
  else:
    m = jnp.max(z, axis=1, keepdims=True)
    lse = jnp.log(jnp.sum(jnp.exp(z - m), axis=1, keepdims=True)) + m
    o_ref[...] = z - lse


def _tc_call(s_parts, deg_parts, x, wl, bl, wr, bn, last):
  n, d = x.shape
  npad = deg_parts.shape[1]
  grid = ((n + bn - 1) // bn,)
  row_spec = pl.BlockSpec((bn, d), lambda i: (i, 0))
  deg_spec = pl.BlockSpec((NW, npad), lambda i: (0, 0))
  w_spec = pl.BlockSpec((d, d), lambda i: (0, 0))
  b_spec = pl.BlockSpec((1, d), lambda i: (0, 0))
  return pl.pallas_call(
      functools.partial(_tc_layer, bn=bn, last=last),
      grid=grid,
      in_specs=[row_spec, row_spec, deg_spec, row_spec,
                w_spec, b_spec, w_spec],
      out_specs=row_spec,
      out_shape=jax.ShapeDtypeStruct((n, d), jnp.float32),
  )(s_parts[0], s_parts[1], deg_parts, x, wl, bl.reshape(1, d), wr)


@jax.jit
def kernel(x, edge_index, Wl1, bl1, Wr1, Wl2, bl2, Wr2):
  n, d = x.shape
  e = edge_index.shape[1]
  epb = BR * CHUNK
  assert e % (NW * epb) == 0
  nb = e // (NW * epb)
  bn = 1024
  npad = ((n + bn - 1) // bn) * bn
  assert npad % (8 * NS) == 0

  ei = edge_index.astype(jnp.int32)
  src4 = ei[0].reshape(NW, nb, BR, CHUNK)
  dst4 = ei[1].reshape(NW, nb, BR, CHUNK)

  zrows = jnp.zeros((CHUNK, d), jnp.float32)
  zdeg = jnp.zeros((npad,), jnp.float32)

  sc_l1 = _make_sc_segment_sum(n, npad, d, nb, with_deg=True)
  sc_l2 = _make_sc_segment_sum(n, npad, d, nb, with_deg=False)

  s1_parts, deg_parts = sc_l1(x, src4, dst4, zrows, zdeg)
  h = _tc_call(s1_parts, deg_parts, x, Wl1, bl1, Wr1, bn, last=False)

  (s2_parts,) = sc_l2(h, src4, dst4, zrows, zdeg)
  out = _tc_call(s2_parts, deg_parts, h, Wl2, bl2, Wr2, bn, last=True)
  return out


# speedup vs baseline: 1.2224x; 1.0275x over previous
import functools

import jax
import jax.numpy as jnp
from jax import lax
from jax.experimental import pallas as pl
from jax.experimental.pallas import tpu as pltpu
from jax.experimental.pallas import tpu_sc as plsc

NC = 2
NS = 16
NW = NC * NS
L = 16
CHUNK = 80
BR = 25


def _make_sc_segment_sum(n, npad, d, nb, with_deg):
  rpt = npad // NS

  mesh = plsc.VectorSubcoreMesh(core_axis_name="c", subcore_axis_name="s")

  out_type = [jax.ShapeDtypeStruct((NC, npad, d), jnp.float32)]
  if with_deg:
    out_type.append(jax.ShapeDtypeStruct((NW, npad), jnp.float32))

  assert rpt % CHUNK == 0

  scratch = [
      pltpu.VMEM((2, BR, CHUNK), jnp.int32),
      pltpu.VMEM((2, BR, CHUNK), jnp.int32),
      pltpu.VMEM((CHUNK, d), jnp.float32),
      pltpu.VMEM((CHUNK, d), jnp.float32),
      pltpu.VMEM_SHARED((npad, d), jnp.float32),
      pltpu.SemaphoreType.DMA,
      pltpu.SemaphoreType.DMA,
      pltpu.SemaphoreType.DMA,
      pltpu.SemaphoreType.DMA,
  ]
  if with_deg:
    scratch.append(pltpu.VMEM((npad,), jnp.float32))

  def body(x_hbm, src_hbm, dst_hbm, zrows_hbm, zdeg_hbm,
           s_out, deg_out, src_v, dst_v, rows_v0, rows_v1, acc_sh,
           gsem0, gsem1, isem, xsem, degp):
    c = lax.axis_index("c")
    s = lax.axis_index("s")
    wid = c * NS + s

    pltpu.sync_copy(zrows_hbm, rows_v0)

    def zstep(r, carry):
      pltpu.sync_copy(rows_v0, acc_sh.at[pl.ds(s * rpt + r * CHUNK, CHUNK)])
      return carry

    lax.fori_loop(0, rpt // CHUNK, zstep, 0)
    if with_deg:
      pltpu.sync_copy(zdeg_hbm, degp)

    plsc.subcore_barrier()

    ones16 = jnp.full((L,), 1.0, jnp.float32)

    pltpu.sync_copy(src_hbm.at[wid, 0], src_v.at[0])
    pltpu.sync_copy(dst_hbm.at[wid, 0], dst_v.at[0])

    for b in range(nb):
      sv = src_v.at[b % 2]
      dv = dst_v.at[b % 2]
      if b + 1 < nb:
        pltpu.async_copy(src_hbm.at[wid, b + 1], src_v.at[(b + 1) % 2], isem)
        pltpu.async_copy(dst_hbm.at[wid, b + 1], dst_v.at[(b + 1) % 2], isem)

      def deg_add(j, dv=dv):
        if with_deg:
          for k in range(CHUNK // L):
            idx = dv[j, pl.ds(k * L, L)]
            plsc.addupdate_scatter(degp, [idx], ones16)

      pltpu.async_copy(x_hbm.at[sv.at[0]], rows_v0, gsem0)

      def pair(j2, carry2, sv=sv, dv=dv, deg_add=deg_add):
        j = 2 * j2

        @pl.when(j + 1 < BR)
        def _():
          pltpu.async_copy(x_hbm.at[sv.at[j + 1]], rows_v1, gsem1)

        pltpu.make_async_copy(x_hbm.at[sv.at[j]], rows_v0, gsem0).wait()
        pltpu.sync_copy(rows_v0, acc_sh.at[dv.at[j]], add=True)
        deg_add(j)

        @pl.when(j + 2 < BR)
        def _():
          pltpu.async_copy(x_hbm.at[sv.at[j + 2]], rows_v0, gsem0)

        @pl.when(j + 1 < BR)
        def _():
          pltpu.make_async_copy(
              x_hbm.at[sv.at[j + 1]], rows_v1, gsem1).wait()
          pltpu.sync_copy(rows_v1, acc_sh.at[dv.at[j + 1]], add=True)
          deg_add(j + 1)

        return carry2

      lax.fori_loop(0, (BR + 1) // 2, pair, 0)

      if b + 1 < nb:
        pltpu.make_async_copy(
            src_hbm.at[wid, b + 1], src_v.at[(b + 1) % 2], isem).wait()
        pltpu.make_async_copy(
            dst_hbm.at[wid, b + 1], dst_v.at[(b + 1) % 2], isem).wait()

    if with_deg:
      pltpu.sync_copy(degp, deg_out.at[wid])

    plsc.subcore_barrier()

    pltpu.sync_copy(acc_sh.at[pl.ds(s * rpt, rpt)],
                    s_out.at[c, pl.ds(s * rpt, rpt)])

  cp = pltpu.CompilerParams(needs_layout_passes=False)
  if with_deg:
    def body_wd(x_hbm, src_hbm, dst_hbm, zr, zd, s_out, deg_out,
                src_v, dst_v, rows_v0, rows_v1, acc_sh,
                gsem0, gsem1, isem, xsem, degp):
      body(x_hbm, src_hbm, dst_hbm, zr, zd, s_out, deg_out,
           src_v, dst_v, rows_v0, rows_v1, acc_sh,
           gsem0, gsem1, isem, xsem, degp)
    fn = pl.kernel(body_wd, out_type=tuple(out_type), mesh=mesh,
                   compiler_params=cp, scratch_types=scratch)
  else:
    def body_nd(x_hbm, src_hbm, dst_hbm, zr, zd, s_out,
                src_v, dst_v, rows_v0, rows_v1, acc_sh,
                gsem0, gsem1, isem, xsem):
      body(x_hbm, src_hbm, dst_hbm, zr, zd, s_out, None,
           src_v, dst_v, rows_v0, rows_v1, acc_sh,
           gsem0, gsem1, isem, xsem, None)
    fn = pl.kernel(body_nd, out_type=tuple(out_type), mesh=mesh,
                   compiler_params=cp, scratch_types=scratch)
  return fn


def _tc_layer(s0_ref, s1_ref, d_ref, x_ref, wl_ref, bl_ref, wr_ref, o_ref,
              *, bn, last):
  pid = pl.program_id(0)
  deg = jnp.sum(d_ref[:, pl.ds(pid * bn, bn)], axis=0)[:, None]
  agg = (s0_ref[...] + s1_ref[...]) / jnp.maximum(deg, 1.0)
  z = lax.dot_general(agg, wl_ref[...], (((1,), (1,)), ((), ())),
                      preferred_element_type=jnp.float32)
  z = z + bl_ref[...]
  z = z + lax.dot_general(x_ref[...], wr_ref[...], (((1,), (1,)), ((), ())),
                          preferred_element_type=jnp.float32)
  if not last:
    o_ref[...] = jnp.maximum(z, 0.0)
  else:
    m = jnp.max(z, axis=1, keepdims=True)
    lse = jnp.log(jnp.sum(jnp.exp(z - m), axis=1, keepdims=True)) + m
    o_ref[...] = z - lse


def _tc_call(s_parts, deg_parts, x, wl, bl, wr, bn, last):
  n, d = x.shape
  npad = deg_parts.shape[1]
  grid = ((n + bn - 1) // bn,)
  row_spec = pl.BlockSpec((bn, d), lambda i: (i, 0))
  deg_spec = pl.BlockSpec((NW, npad), lambda i: (0, 0))
  w_spec = pl.BlockSpec((d, d), lambda i: (0, 0))
  b_spec = pl.BlockSpec((1, d), lambda i: (0, 0))
  return pl.pallas_call(
      functools.partial(_tc_layer, bn=bn, last=last),
      grid=grid,
      in_specs=[row_spec, row_spec, deg_spec, row_spec,
                w_spec, b_spec, w_spec],
      out_specs=row_spec,
      out_shape=jax.ShapeDtypeStruct((n, d), jnp.float32),
  )(s_parts[0], s_parts[1], deg_parts, x, wl, bl.reshape(1, d), wr)


@jax.jit
def kernel(x, edge_index, Wl1, bl1, Wr1, Wl2, bl2, Wr2):
  n, d = x.shape
  e = edge_index.shape[1]
  epb = BR * CHUNK
  assert e % (NW * epb) == 0
  nb = e // (NW * epb)
  bn = 1024
  npad = ((n + bn - 1) // bn) * bn
  assert npad % (8 * NS) == 0

  ei = edge_index.astype(jnp.int32)
  src4 = ei[0].reshape(NW, nb, BR, CHUNK)
  dst4 = ei[1].reshape(NW, nb, BR, CHUNK)

  zrows = jnp.zeros((CHUNK, d), jnp.float32)
  zdeg = jnp.zeros((npad,), jnp.float32)

  sc_l1 = _make_sc_segment_sum(n, npad, d, nb, with_deg=True)
  sc_l2 = _make_sc_segment_sum(n, npad, d, nb, with_deg=False)

  s1_parts, deg_parts = sc_l1(x, src4, dst4, zrows, zdeg)
  h = _tc_call(s1_parts, deg_parts, x, Wl1, bl1, Wr1, bn, last=False)

  (s2_parts,) = sc_l2(h, src4, dst4, zrows, zdeg)
  out = _tc_call(s2_parts, deg_parts, h, Wl2, bl2, Wr2, bn, last=True)
  return out


# speedup vs baseline: 1.2393x; 1.0139x over previous
import functools

import jax
import jax.numpy as jnp
from jax import lax
from jax.experimental import pallas as pl
from jax.experimental.pallas import tpu as pltpu
from jax.experimental.pallas import tpu_sc as plsc

NC = 2
NS = 16
NW = NC * NS
L = 16
CHUNK = 80
BR = 25


def _make_sc_segment_sum(n, npad, d, nb, with_deg):
  rpt = npad // NS

  mesh = plsc.VectorSubcoreMesh(core_axis_name="c", subcore_axis_name="s")

  out_type = [jax.ShapeDtypeStruct((NC, npad, d), jnp.float32)]
  if with_deg:
    out_type.append(jax.ShapeDtypeStruct((NW, npad), jnp.float32))

  assert rpt % CHUNK == 0

  scratch = [
      pltpu.VMEM((2, BR, CHUNK), jnp.int32),
      pltpu.VMEM((2, BR, CHUNK), jnp.int32),
      pltpu.VMEM((CHUNK, d), jnp.float32),
      pltpu.VMEM((CHUNK, d), jnp.float32),
      pltpu.VMEM_SHARED((npad, d), jnp.float32),
      pltpu.SemaphoreType.DMA,
      pltpu.SemaphoreType.DMA,
      pltpu.SemaphoreType.DMA,
      pltpu.SemaphoreType.DMA,
  ]
  if with_deg:
    scratch.append(pltpu.VMEM((npad,), jnp.float32))

  def body(x_hbm, src_hbm, dst_hbm, zrows_hbm, zdeg_hbm,
           s_out, deg_out, src_v, dst_v, rows_v0, rows_v1, acc_sh,
           gsem0, gsem1, isem, xsem, degp):
    c = lax.axis_index("c")
    s = lax.axis_index("s")
    wid = c * NS + s

    pltpu.sync_copy(zrows_hbm, rows_v0)

    def zstep(r, carry):
      pltpu.sync_copy(rows_v0, acc_sh.at[pl.ds(s * rpt + r * CHUNK, CHUNK)])
      return carry

    lax.fori_loop(0, rpt // CHUNK, zstep, 0)
    if with_deg:
      pltpu.sync_copy(zdeg_hbm, degp)

    plsc.subcore_barrier()

    ones16 = jnp.full((L,), 1.0, jnp.float32)

    pltpu.sync_copy(src_hbm.at[wid, 0], src_v.at[0])
    pltpu.sync_copy(dst_hbm.at[wid, 0], dst_v.at[0])

    for b in range(nb):
      sv = src_v.at[b % 2]
      dv = dst_v.at[b % 2]
      if b + 1 < nb:
        pltpu.async_copy(src_hbm.at[wid, b + 1], src_v.at[(b + 1) % 2], isem)
        pltpu.async_copy(dst_hbm.at[wid, b + 1], dst_v.at[(b + 1) % 2], isem)

      def deg_add(j, dv=dv):
        if with_deg:
          for k in range(CHUNK // L):
            idx = dv[j, pl.ds(k * L, L)]
            plsc.addupdate_scatter(degp, [idx], ones16)

      pltpu.async_copy(x_hbm.at[sv.at[0]], rows_v0, gsem0)

      def pair(j2, carry2, sv=sv, dv=dv, deg_add=deg_add):
        j = 2 * j2

        @pl.when(j + 1 < BR)
        def _():
          pltpu.async_copy(x_hbm.at[sv.at[j + 1]], rows_v1, gsem1)

        pltpu.make_async_copy(x_hbm.at[sv.at[j]], rows_v0, gsem0).wait()
        pltpu.sync_copy(rows_v0, acc_sh.at[dv.at[j]], add=True)
        deg_add(j)

        @pl.when(j + 2 < BR)
        def _():
          pltpu.async_copy(x_hbm.at[sv.at[j + 2]], rows_v0, gsem0)

        @pl.when(j + 1 < BR)
        def _():
          pltpu.make_async_copy(
              x_hbm.at[sv.at[j + 1]], rows_v1, gsem1).wait()
          pltpu.sync_copy(rows_v1, acc_sh.at[dv.at[j + 1]], add=True)
          deg_add(j + 1)

        return carry2

      lax.fori_loop(0, (BR + 1) // 2, pair, 0)

      if b + 1 < nb:
        pltpu.make_async_copy(
            src_hbm.at[wid, b + 1], src_v.at[(b + 1) % 2], isem).wait()
        pltpu.make_async_copy(
            dst_hbm.at[wid, b + 1], dst_v.at[(b + 1) % 2], isem).wait()

    if with_deg:
      pltpu.sync_copy(degp, deg_out.at[wid])

    plsc.subcore_barrier()

    pltpu.sync_copy(acc_sh.at[pl.ds(s * rpt, rpt)],
                    s_out.at[c, pl.ds(s * rpt, rpt)])

  cp = pltpu.CompilerParams(needs_layout_passes=False)
  if with_deg:
    def body_wd(x_hbm, src_hbm, dst_hbm, zr, zd, s_out, deg_out,
                src_v, dst_v, rows_v0, rows_v1, acc_sh,
                gsem0, gsem1, isem, xsem, degp):
      body(x_hbm, src_hbm, dst_hbm, zr, zd, s_out, deg_out,
           src_v, dst_v, rows_v0, rows_v1, acc_sh,
           gsem0, gsem1, isem, xsem, degp)
    fn = pl.kernel(body_wd, out_type=tuple(out_type), mesh=mesh,
                   compiler_params=cp, scratch_types=scratch)
  else:
    def body_nd(x_hbm, src_hbm, dst_hbm, zr, zd, s_out,
                src_v, dst_v, rows_v0, rows_v1, acc_sh,
                gsem0, gsem1, isem, xsem):
      body(x_hbm, src_hbm, dst_hbm, zr, zd, s_out, None,
           src_v, dst_v, rows_v0, rows_v1, acc_sh,
           gsem0, gsem1, isem, xsem, None)
    fn = pl.kernel(body_nd, out_type=tuple(out_type), mesh=mesh,
                   compiler_params=cp, scratch_types=scratch)
  return fn


def _tc_layer(s0_ref, s1_ref, d_ref, x_ref, wl_ref, bl_ref, wr_ref, o_ref,
              *, bn, last):
  pid = pl.program_id(0)
  deg = jnp.sum(d_ref[:, pl.ds(pid * bn, bn)], axis=0)[:, None]
  agg = (s0_ref[...] + s1_ref[...]) / jnp.maximum(deg, 1.0)
  z = lax.dot_general(agg, wl_ref[...], (((1,), (1,)), ((), ())),
                      preferred_element_type=jnp.float32)
  z = z + bl_ref[...]
  z = z + lax.dot_general(x_ref[...], wr_ref[...], (((1,), (1,)), ((), ())),
                          preferred_element_type=jnp.float32)
  if not last:
    o_ref[...] = jnp.maximum(z, 0.0)
  else:
    m = jnp.max(z, axis=1, keepdims=True)
    lse = jnp.log(jnp.sum(jnp.exp(z - m), axis=1, keepdims=True)) + m
    o_ref[...] = z - lse


def _tc_call(s_parts, deg_parts, x, wl, bl, wr, bn, last):
  n, d = x.shape
  npad = deg_parts.shape[1]
  grid = ((n + bn - 1) // bn,)
  row_spec = pl.BlockSpec((bn, d), lambda i: (i, 0))
  deg_spec = pl.BlockSpec((NW, npad), lambda i: (0, 0))
  w_spec = pl.BlockSpec((d, d), lambda i: (0, 0))
  b_spec = pl.BlockSpec((1, d), lambda i: (0, 0))
  return pl.pallas_call(
      functools.partial(_tc_layer, bn=bn, last=last),
      grid=grid,
      in_specs=[row_spec, row_spec, deg_spec, row_spec,
                w_spec, b_spec, w_spec],
      out_specs=row_spec,
      out_shape=jax.ShapeDtypeStruct((n, d), jnp.float32),
  )(s_parts[0], s_parts[1], deg_parts, x, wl, bl.reshape(1, d), wr)


@jax.jit
def kernel(x, edge_index, Wl1, bl1, Wr1, Wl2, bl2, Wr2):
  n, d = x.shape
  e = edge_index.shape[1]
  epb = BR * CHUNK
  assert e % (NW * epb) == 0
  nb = e // (NW * epb)
  bn = 2048
  npad = ((n + bn - 1) // bn) * bn
  assert npad % (8 * NS) == 0

  ei = edge_index.astype(jnp.int32)
  src4 = ei[0].reshape(NW, nb, BR, CHUNK)
  dst4 = ei[1].reshape(NW, nb, BR, CHUNK)

  zrows = jnp.zeros((CHUNK, d), jnp.float32)
  zdeg = jnp.zeros((npad,), jnp.float32)

  sc_l1 = _make_sc_segment_sum(n, npad, d, nb, with_deg=True)
  sc_l2 = _make_sc_segment_sum(n, npad, d, nb, with_deg=False)

  s1_parts, deg_parts = sc_l1(x, src4, dst4, zrows, zdeg)
  h = _tc_call(s1_parts, deg_parts, x, Wl1, bl1, Wr1, bn, last=False)

  (s2_parts,) = sc_l2(h, src4, dst4, zrows, zdeg)
  out = _tc_call(s2_parts, deg_parts, h, Wl2, bl2, Wr2, bn, last=True)
  return out


# speedup vs baseline: 1.2508x; 1.0093x over previous
import functools

import jax
import jax.numpy as jnp
from jax import lax
from jax.experimental import pallas as pl
from jax.experimental.pallas import tpu as pltpu
from jax.experimental.pallas import tpu_sc as plsc

NC = 2
NS = 16
NW = NC * NS
L = 16
CHUNK = 80
BR = 25


def _make_sc_segment_sum(n, npad, d, nb, with_deg):
  rpt = npad // NS

  mesh = plsc.VectorSubcoreMesh(core_axis_name="c", subcore_axis_name="s")

  out_type = [jax.ShapeDtypeStruct((NC, npad, d), jnp.float32)]
  if with_deg:
    out_type.append(jax.ShapeDtypeStruct((NW, npad), jnp.float32))

  assert rpt % CHUNK == 0

  scratch = [
      pltpu.VMEM((2, BR, CHUNK), jnp.int32),
      pltpu.VMEM((2, BR, CHUNK), jnp.int32),
      pltpu.VMEM((CHUNK, d), jnp.float32),
      pltpu.VMEM((CHUNK, d), jnp.float32),
      pltpu.VMEM_SHARED((npad, d), jnp.float32),
      pltpu.SemaphoreType.DMA,
      pltpu.SemaphoreType.DMA,
      pltpu.SemaphoreType.DMA,
      pltpu.SemaphoreType.DMA,
  ]
  if with_deg:
    scratch.append(pltpu.VMEM((npad,), jnp.float32))

  def body(x_hbm, src_hbm, dst_hbm, zrows_hbm, zdeg_hbm,
           s_out, deg_out, src_v, dst_v, rows_v0, rows_v1, acc_sh,
           gsem0, gsem1, isem, xsem, degp):
    c = lax.axis_index("c")
    s = lax.axis_index("s")
    wid = c * NS + s

    pltpu.sync_copy(zrows_hbm, rows_v0)

    def zstep(r, carry):
      pltpu.sync_copy(rows_v0, acc_sh.at[pl.ds(s * rpt + r * CHUNK, CHUNK)])
      return carry

    lax.fori_loop(0, rpt // CHUNK, zstep, 0)
    if with_deg:
      pltpu.sync_copy(zdeg_hbm, degp)

    plsc.subcore_barrier()

    ones16 = jnp.full((L,), 1.0, jnp.float32)

    pltpu.sync_copy(src_hbm.at[wid, 0], src_v.at[0])
    pltpu.sync_copy(dst_hbm.at[wid, 0], dst_v.at[0])

    for b in range(nb):
      sv = src_v.at[b % 2]
      dv = dst_v.at[b % 2]
      if b + 1 < nb:
        pltpu.async_copy(src_hbm.at[wid, b + 1], src_v.at[(b + 1) % 2], isem)
        pltpu.async_copy(dst_hbm.at[wid, b + 1], dst_v.at[(b + 1) % 2], isem)

      def deg_add(j, dv=dv):
        if with_deg:
          for k in range(CHUNK // L):
            idx = dv[j, pl.ds(k * L, L)]
            plsc.addupdate_scatter(degp, [idx], ones16)

      pltpu.async_copy(x_hbm.at[sv.at[0]], rows_v0, gsem0)

      def pair(j2, carry2, sv=sv, dv=dv, deg_add=deg_add):
        j = 2 * j2

        @pl.when(j + 1 < BR)
        def _():
          pltpu.async_copy(x_hbm.at[sv.at[j + 1]], rows_v1, gsem1)

        pltpu.make_async_copy(x_hbm.at[sv.at[j]], rows_v0, gsem0).wait()
        pltpu.sync_copy(rows_v0, acc_sh.at[dv.at[j]], add=True)
        deg_add(j)

        @pl.when(j + 2 < BR)
        def _():
          pltpu.async_copy(x_hbm.at[sv.at[j + 2]], rows_v0, gsem0)

        @pl.when(j + 1 < BR)
        def _():
          pltpu.make_async_copy(
              x_hbm.at[sv.at[j + 1]], rows_v1, gsem1).wait()
          pltpu.sync_copy(rows_v1, acc_sh.at[dv.at[j + 1]], add=True)
          deg_add(j + 1)

        return carry2

      lax.fori_loop(0, (BR + 1) // 2, pair, 0)

      if b + 1 < nb:
        pltpu.make_async_copy(
            src_hbm.at[wid, b + 1], src_v.at[(b + 1) % 2], isem).wait()
        pltpu.make_async_copy(
            dst_hbm.at[wid, b + 1], dst_v.at[(b + 1) % 2], isem).wait()

    if with_deg:
      pltpu.sync_copy(degp, deg_out.at[wid])

    plsc.subcore_barrier()

    pltpu.sync_copy(acc_sh.at[pl.ds(s * rpt, rpt)],
                    s_out.at[c, pl.ds(s * rpt, rpt)])

  cp = pltpu.CompilerParams(needs_layout_passes=False)
  if with_deg:
    def body_wd(x_hbm, src_hbm, dst_hbm, zr, zd, s_out, deg_out,
                src_v, dst_v, rows_v0, rows_v1, acc_sh,
                gsem0, gsem1, isem, xsem, degp):
      body(x_hbm, src_hbm, dst_hbm, zr, zd, s_out, deg_out,
           src_v, dst_v, rows_v0, rows_v1, acc_sh,
           gsem0, gsem1, isem, xsem, degp)
    fn = pl.kernel(body_wd, out_type=tuple(out_type), mesh=mesh,
                   compiler_params=cp, scratch_types=scratch)
  else:
    def body_nd(x_hbm, src_hbm, dst_hbm, zr, zd, s_out,
                src_v, dst_v, rows_v0, rows_v1, acc_sh,
                gsem0, gsem1, isem, xsem):
      body(x_hbm, src_hbm, dst_hbm, zr, zd, s_out, None,
           src_v, dst_v, rows_v0, rows_v1, acc_sh,
           gsem0, gsem1, isem, xsem, None)
    fn = pl.kernel(body_nd, out_type=tuple(out_type), mesh=mesh,
                   compiler_params=cp, scratch_types=scratch)
  return fn


def _tc_layer(s0_ref, s1_ref, d_ref, x_ref, wl_ref, bl_ref, wr_ref, o_ref,
              *, bn, last):
  pid = pl.program_id(0)
  deg = jnp.sum(d_ref[:, pl.ds(pid * bn, bn)], axis=0)[:, None]
  agg = (s0_ref[...] + s1_ref[...]) / jnp.maximum(deg, 1.0)
  z = lax.dot_general(agg, wl_ref[...], (((1,), (1,)), ((), ())),
                      preferred_element_type=jnp.float32)
  z = z + bl_ref[...]
  z = z + lax.dot_general(x_ref[...], wr_ref[...], (((1,), (1,)), ((), ())),
                          preferred_element_type=jnp.float32)
  if not last:
    o_ref[...] = jnp.maximum(z, 0.0)
  else:
    m = jnp.max(z, axis=1, keepdims=True)
    lse = jnp.log(jnp.sum(jnp.exp(z - m), axis=1, keepdims=True)) + m
    o_ref[...] = z - lse


def _tc_call(s_parts, deg_parts, x, wl, bl, wr, bn, last):
  n, d = x.shape
  npad = deg_parts.shape[1]
  grid = ((n + bn - 1) // bn,)
  row_spec = pl.BlockSpec((bn, d), lambda i: (i, 0))
  deg_spec = pl.BlockSpec((NW, npad), lambda i: (0, 0))
  w_spec = pl.BlockSpec((d, d), lambda i: (0, 0))
  b_spec = pl.BlockSpec((1, d), lambda i: (0, 0))
  return pl.pallas_call(
      functools.partial(_tc_layer, bn=bn, last=last),
      grid=grid,
      in_specs=[row_spec, row_spec, deg_spec, row_spec,
                w_spec, b_spec, w_spec],
      out_specs=row_spec,
      out_shape=jax.ShapeDtypeStruct((n, d), jnp.float32),
  )(s_parts[0], s_parts[1], deg_parts, x, wl, bl.reshape(1, d), wr)


@jax.jit
def kernel(x, edge_index, Wl1, bl1, Wr1, Wl2, bl2, Wr2):
  n, d = x.shape
  e = edge_index.shape[1]
  epb = BR * CHUNK
  assert e % (NW * epb) == 0
  nb = e // (NW * epb)
  bn = 5120
  npad = ((n + bn - 1) // bn) * bn
  assert npad % (8 * NS) == 0

  ei = edge_index.astype(jnp.int32)
  src4 = ei[0].reshape(NW, nb, BR, CHUNK)
  dst4 = ei[1].reshape(NW, nb, BR, CHUNK)

  zrows = jnp.zeros((CHUNK, d), jnp.float32)
  zdeg = jnp.zeros((npad,), jnp.float32)

  sc_l1 = _make_sc_segment_sum(n, npad, d, nb, with_deg=True)
  sc_l2 = _make_sc_segment_sum(n, npad, d, nb, with_deg=False)

  s1_parts, deg_parts = sc_l1(x, src4, dst4, zrows, zdeg)
  h = _tc_call(s1_parts, deg_parts, x, Wl1, bl1, Wr1, bn, last=False)

  (s2_parts,) = sc_l2(h, src4, dst4, zrows, zdeg)
  out = _tc_call(s2_parts, deg_parts, h, Wl2, bl2, Wr2, bn, last=True)
  return out
